# baseline (speedup 1.0000x reference)
import functools

import jax
import jax.numpy as jnp
from jax import lax
from jax.experimental import pallas as pl
from jax.experimental.pallas import tpu as pltpu
from jax.experimental.pallas import tpu_sc as plsc

EMB_DIM = 64
SCALE = 8.0
LANES = 16
CHUNK = 128
DEPTH = 4


def _emb_call(n_tokens, per_w, num_cores):
    n_chunks = per_w // CHUNK
    n_groups = n_chunks // DEPTH
    mesh = plsc.VectorSubcoreMesh(core_axis_name="c", subcore_axis_name="s")

    buf = lambda: pltpu.VMEM((CHUNK, EMB_DIM), jnp.float32)

    @functools.partial(
        pl.kernel,
        mesh=mesh,
        out_type=jax.ShapeDtypeStruct((n_tokens, EMB_DIM), jnp.float32),
        compiler_params=pltpu.CompilerParams(use_tc_tiling_on_sc=False),
        scratch_types=(
            [pltpu.VMEM((per_w,), jnp.int32)]
            + [buf() for _ in range(2 * DEPTH)]
            + [pltpu.SemaphoreType.DMA for _ in range(2 * DEPTH)]
        ),
    )
    def emb_k(idx_hbm, tab_hbm, out_hbm, idx_v, *bufs):
        ins = bufs[0:DEPTH]
        outs = bufs[DEPTH:2 * DEPTH]
        gsems = bufs[2 * DEPTH:3 * DEPTH]
        wsems = bufs[3 * DEPTH:4 * DEPTH]

        wid = lax.axis_index("s") * num_cores + lax.axis_index("c")
        base = wid * per_w
        pltpu.sync_copy(idx_hbm.at[pl.ds(base, per_w)], idx_v)

        def gather(c, b):
            pltpu.async_copy(
                tab_hbm.at[idx_v.at[pl.ds(c * CHUNK, CHUNK)]],
                ins[b], gsems[b]
            )

        def gather_wait(b):
            pltpu.make_async_copy(
                tab_hbm.at[idx_v.at[pl.ds(0, CHUNK)]], ins[b], gsems[b]
            ).wait()

        def write(c, b):
            pltpu.async_copy(
                outs[b], out_hbm.at[pl.ds(base + c * CHUNK, CHUNK)], wsems[b]
            )

        def write_wait(b):
            pltpu.make_async_copy(
                outs[b], out_hbm.at[pl.ds(base, CHUNK)], wsems[b]
            ).wait()

        for b in range(DEPTH):
            gather(b, b)

        def group_body(g, carry):
            for b in range(DEPTH):
                c = g * DEPTH + b
                gather_wait(b)

                @pl.when(g >= 1)
                def _():
                    write_wait(b)

                src, dst = ins[b], outs[b]

                def row_body(r8, carry2):
                    r0 = r8 * 8
                    for k in range(8):
                        for j in range(EMB_DIM // LANES):
                            sl = pl.ds(j * LANES, LANES)
                            dst[r0 + k, sl] = src[r0 + k, sl] * SCALE
                    return carry2

                lax.fori_loop(0, CHUNK // 8, row_body, 0)

                write(c, b)

                @pl.when(g < n_groups - 1)
                def _():
                    gather(c + DEPTH, b)

            return carry

        lax.fori_loop(0, n_groups, group_body, 0)
        for b in range(DEPTH):
            write_wait(b)

    return emb_k


def kernel(x, table):
    b, l = x.shape
    n_tokens = b * l
    info = plsc.get_sparse_core_info()
    n_workers = info.num_cores * info.num_subcores
    per_w = n_tokens // n_workers
    emb_k = _emb_call(n_tokens, per_w, info.num_cores)
    return emb_k(x.reshape(n_tokens), table).reshape(b, l, EMB_DIM)
